# Initial kernel scaffold; baseline (speedup 1.0000x reference)
import jax
import jax.numpy as jnp
from jax.experimental import pallas as pl


def kernel(x, edge_index, W_enc, b_enc, W1, b1, W2, b2, W_dec, b_dec):
    raise NotImplementedError("write your pallas kernel here")


# speedup vs baseline: 3.0580x; 3.0580x over previous
import jax
import jax.numpy as jnp
from jax import lax
from jax.experimental import pallas as pl
from jax.experimental.pallas import tpu as pltpu
from jax.experimental.pallas import tpu_sc as plsc

N = 10000
E = 320000
D_IN = 128
D_H = 256
D_OUT = 128

_NSUB = 16
_CHUNK = 128
_EROWS = 2560
_EPAD = _EROWS * _CHUNK
_ROWS_PER_SUB = _EROWS // _NSUB
_G = 64
_NACC = N + 8
_NODE_ROWS = 624
_NODE_EXTRA = N - _NSUB * _NODE_ROWS
_ZC = 104
_PREC = lax.Precision.HIGHEST


def _make_sc_scatter(with_count: bool):
    out_type = [jax.ShapeDtypeStruct((N, _G), jnp.float32)] * 4
    if with_count:
        out_type = out_type + [jax.ShapeDtypeStruct((N, 16), jnp.float32)]

    scratch = [
        pltpu.VMEM((_ROWS_PER_SUB, _CHUNK), jnp.int32),
        pltpu.VMEM((_ROWS_PER_SUB, _CHUNK), jnp.int32),
        pltpu.VMEM((2, _CHUNK, _G), jnp.float32),
        pltpu.VMEM((_CHUNK, 16), jnp.float32),
        pltpu.VMEM_SHARED((_NACC, _G), jnp.float32),
        pltpu.SemaphoreType.DMA,
    ]
    if with_count:
        scratch.append(pltpu.VMEM_SHARED((_NACC, 16), jnp.float32))

    mesh = plsc.VectorSubcoreMesh(core_axis_name="c", subcore_axis_name="s")

    def body(t0, t1, t2, t3, src2d, dst2d, *rest):
        if with_count:
            (o0, o1, o2, o3, ocnt,
             srcbuf, dstbuf, rb, ones16, acc, sem, cacc) = rest
        else:
            o0, o1, o2, o3, srcbuf, dstbuf, rb, ones16, acc, sem = rest
            ocnt = cacc = None

        cid = lax.axis_index("c")
        sid = lax.axis_index("s")
        zrow = jnp.zeros((16,), jnp.float32)
        last = sid == _NSUB - 1
        r0 = sid * _NODE_ROWS

        def zero_rb():
            def zi(i, _):
                def zj(j, _):
                    rb[0, i, pl.ds(j * 16, 16)] = zrow
                    rb[1, i, pl.ds(j * 16, 16)] = zrow
                    return 0
                return lax.fori_loop(0, _G // 16, zj, 0)
            lax.fori_loop(0, _CHUNK, zi, 0)

        def fill_16wide(ref, val):
            v = jnp.full((16,), val, jnp.float32)
            def fi(i, _):
                ref[i, :] = v
                return 0
            lax.fori_loop(0, _CHUNK, fi, 0)

        def zero_shared(a_ref, src_ref):
            def zk(k, _):
                pltpu.sync_copy(src_ref.at[pl.ds(0, _ZC)],
                                a_ref.at[pl.ds(r0 + k * _ZC, _ZC)])
                return 0
            lax.fori_loop(0, _NODE_ROWS // _ZC, zk, 0)
            @pl.when(last)
            def _():
                pltpu.sync_copy(src_ref.at[pl.ds(0, _NODE_EXTRA)],
                                a_ref.at[pl.ds(_NSUB * _NODE_ROWS, _NODE_EXTRA)])

        def writeback(a_ref, out_ref):
            pltpu.sync_copy(a_ref.at[pl.ds(r0, _NODE_ROWS)],
                            out_ref.at[pl.ds(r0, _NODE_ROWS)])
            @pl.when(last)
            def _():
                es = pl.ds(_NSUB * _NODE_ROWS, _NODE_EXTRA)
                pltpu.sync_copy(a_ref.at[es], out_ref.at[es])

        def phase(t_ref, out_ref, count: bool):
            zero_shared(acc, rb.at[0])
            if count:
                fill_16wide(ones16, 0.0)
                zero_shared(cacc, ones16)
                fill_16wide(ones16, 1.0)
            plsc.subcore_barrier()

            dummy = t_ref.at[pl.ds(0, _CHUNK)]
            pltpu.async_copy(t_ref.at[srcbuf.at[0]], rb.at[0], sem)

            def step(j, _):
                b = lax.rem(j, 2)
                pltpu.make_async_copy(dummy, rb.at[b], sem).wait()
                @pl.when(j + 1 < _ROWS_PER_SUB)
                def _():
                    pltpu.async_copy(t_ref.at[srcbuf.at[j + 1]],
                                     rb.at[1 - b], sem)
                pltpu.sync_copy(rb.at[b], acc.at[dstbuf.at[j]], add=True)
                if count:
                    pltpu.sync_copy(ones16, cacc.at[dstbuf.at[j]], add=True)
                return 0
            lax.fori_loop(0, _ROWS_PER_SUB, step, 0)
            plsc.subcore_barrier()

            writeback(acc, out_ref)
            if count:
                writeback(cacc, ocnt)
            zero_rb()
            plsc.subcore_barrier()

        def run_core(t_a, t_b, o_a, o_b, count: bool):
            zero_rb()
            base = sid * _ROWS_PER_SUB
            pltpu.sync_copy(src2d.at[pl.ds(base, _ROWS_PER_SUB)], srcbuf)
            pltpu.sync_copy(dst2d.at[pl.ds(base, _ROWS_PER_SUB)], dstbuf)
            phase(t_a, o_a, count)
            phase(t_b, o_b, False)

        @pl.when(cid == 0)
        def _():
            run_core(t0, t1, o0, o1, with_count)

        @pl.when(cid == 1)
        def _():
            run_core(t2, t3, o2, o3, False)

    return pl.kernel(
        body, out_type=tuple(out_type), mesh=mesh, scratch_types=scratch,
        compiler_params=pltpu.CompilerParams(use_tc_tiling_on_sc=False))


_sc_scatter_count = _make_sc_scatter(True)
_sc_scatter = _make_sc_scatter(False)


_BN = 1000


def _split_store(t, refs):
    for g, r in enumerate(refs):
        r[...] = t[:, g * _G:(g + 1) * _G]


def _tc_enc_body(x_ref, we_ref, be_ref, w1_ref, b1_ref, *t_refs):
    h = jnp.dot(x_ref[...], we_ref[...], precision=_PREC,
                preferred_element_type=jnp.float32) + be_ref[...]
    t = jnp.maximum(jnp.dot(h, w1_ref[...], precision=_PREC,
                            preferred_element_type=jnp.float32) + b1_ref[...],
                    0.0)
    _split_store(t, t_refs)


def _agg_matmul(s_refs, cnt_ref, w2_ref, b2_ref):
    inv = 1.0 / jnp.maximum(cnt_ref[:, 0:1], 1.0)
    h = b2_ref[...]
    for g in range(4):
        h = h + jnp.dot(s_refs[g][...] * inv, w2_ref[g * _G:(g + 1) * _G, :],
                        precision=_PREC, preferred_element_type=jnp.float32)
    return h


def _tc_mid_body(s0, s1, s2, s3, cnt_ref, w2_ref, b2_ref, w1_ref, b1_ref,
                 *t_refs):
    h = _agg_matmul((s0, s1, s2, s3), cnt_ref, w2_ref, b2_ref)
    t = jnp.maximum(jnp.dot(h, w1_ref[...], precision=_PREC,
                            preferred_element_type=jnp.float32) + b1_ref[...],
                    0.0)
    _split_store(t, t_refs)


def _tc_dec_body(s0, s1, s2, s3, cnt_ref, w2_ref, b2_ref, wd_ref, bd_ref,
                 out_ref):
    h = _agg_matmul((s0, s1, s2, s3), cnt_ref, w2_ref, b2_ref)
    out_ref[...] = (jnp.dot(h, wd_ref[...], precision=_PREC,
                            preferred_element_type=jnp.float32) + bd_ref[...])


def _row_spec(w):
    return pl.BlockSpec((_BN, w), lambda i: (i, 0))


def _full_spec(shape):
    return pl.BlockSpec(shape, lambda i: tuple(0 for _ in shape))


_quarter_out = [jax.ShapeDtypeStruct((N, _G), jnp.float32)] * 4
_quarter_specs = [_row_spec(_G)] * 4

_tc_enc = pl.pallas_call(
    _tc_enc_body,
    grid=(N // _BN,),
    in_specs=[_row_spec(D_IN), _full_spec((D_IN, D_H)), _full_spec((1, D_H)),
              _full_spec((D_H, D_H)), _full_spec((1, D_H))],
    out_specs=_quarter_specs,
    out_shape=_quarter_out,
)

_tc_mid = pl.pallas_call(
    _tc_mid_body,
    grid=(N // _BN,),
    in_specs=_quarter_specs + [_row_spec(16),
              _full_spec((D_H, D_H)), _full_spec((1, D_H)),
              _full_spec((D_H, D_H)), _full_spec((1, D_H))],
    out_specs=_quarter_specs,
    out_shape=_quarter_out,
)

_tc_dec = pl.pallas_call(
    _tc_dec_body,
    grid=(N // _BN,),
    in_specs=_quarter_specs + [_row_spec(16),
              _full_spec((D_H, D_H)), _full_spec((1, D_H)),
              _full_spec((D_H, D_OUT)), _full_spec((1, D_OUT))],
    out_specs=_row_spec(D_OUT),
    out_shape=jax.ShapeDtypeStruct((N, D_OUT), jnp.float32),
)


def kernel(x, edge_index, W_enc, b_enc, W1, b1, W2, b2, W_dec, b_dec):
    pad = _EPAD - E
    src2d = jnp.concatenate(
        [edge_index[0], jnp.zeros((pad,), jnp.int32)]).reshape(_EROWS, _CHUNK)
    dst2d = jnp.concatenate(
        [edge_index[1], jnp.full((pad,), N, jnp.int32)]).reshape(_EROWS, _CHUNK)
    be = b_enc.reshape(1, D_H)
    b1r = b1.reshape(1, D_H)
    b2r = b2.reshape(1, D_H)
    bdr = b_dec.reshape(1, D_OUT)

    t = _tc_enc(x, W_enc, be, W1, b1r)
    s0, s1, s2, s3, cnt = _sc_scatter_count(t[0], t[1], t[2], t[3],
                                            src2d, dst2d)
    u = _tc_mid(s0, s1, s2, s3, cnt, W2, b2r, W1, b1r)
    v0, v1, v2, v3 = _sc_scatter(u[0], u[1], u[2], u[3], src2d, dst2d)
    return _tc_dec(v0, v1, v2, v3, cnt, W2, b2r, W_dec, bdr)


# speedup vs baseline: 3.4947x; 1.1428x over previous
import jax
import jax.numpy as jnp
from jax import lax
from jax.experimental import pallas as pl
from jax.experimental.pallas import tpu as pltpu
from jax.experimental.pallas import tpu_sc as plsc

N = 10000
E = 320000
D_IN = 128
D_H = 256
D_OUT = 128

_NSUB = 16
_CHUNK = 128
_EROWS = 2560
_EPAD = _EROWS * _CHUNK
_ROWS_PER_SUB = _EROWS // _NSUB
_G = 64
_NACC = N + 8
_NODE_ROWS = 624
_NODE_EXTRA = N - _NSUB * _NODE_ROWS
_ZC = 104
_CROWS = _EROWS // 32
_PREC = lax.Precision.HIGHEST


def _make_sc_scatter():
    out_type = tuple([jax.ShapeDtypeStruct((N, _G), jnp.float32)] * 4)

    scratch = [
        pltpu.VMEM((_ROWS_PER_SUB, _CHUNK), jnp.int32),
        pltpu.VMEM((_ROWS_PER_SUB, _CHUNK), jnp.int32),
        pltpu.VMEM((4, _CHUNK, _G), jnp.float32),
        pltpu.VMEM((_ZC, _G), jnp.float32),
        pltpu.VMEM_SHARED((_NACC, _G), jnp.float32),
        pltpu.SemaphoreType.DMA,
        pltpu.SemaphoreType.DMA,
    ]

    mesh = plsc.VectorSubcoreMesh(core_axis_name="c", subcore_axis_name="s")

    def body(t0, t1, t2, t3, src2d, dst2d,
             o0, o1, o2, o3, srcbuf, dstbuf, rb, zbuf, acc, semg, sems):
        cid = lax.axis_index("c")
        sid = lax.axis_index("s")
        last = sid == _NSUB - 1
        r0 = sid * _NODE_ROWS

        def phase(t_ref, out_ref):
            _zero_shared(acc, zbuf, r0, last)
            plsc.subcore_barrier()

            dummy_g = t_ref.at[pl.ds(0, _CHUNK)]

            def drain_g(b):
                pltpu.make_async_copy(dummy_g, rb.at[b], semg).wait()

            def drain_s():
                pltpu.make_async_copy(dummy_g, rb.at[0], sems).wait()

            pltpu.async_copy(t_ref.at[srcbuf.at[0]], rb.at[0], semg)
            pltpu.async_copy(t_ref.at[srcbuf.at[1]], rb.at[1], semg)

            def step(j, _):
                b = lax.rem(j, 4)
                drain_g(b)
                pltpu.async_copy(rb.at[b], acc.at[dstbuf.at[j]], sems,
                                 add=True)
                @pl.when(j >= 2)
                def _():
                    drain_s()
                @pl.when(j + 2 < _ROWS_PER_SUB)
                def _():
                    pltpu.async_copy(t_ref.at[srcbuf.at[j + 2]],
                                     rb.at[lax.rem(j + 2, 4)], semg)
                return 0
            lax.fori_loop(0, _ROWS_PER_SUB, step, 0)
            drain_s()
            drain_s()
            plsc.subcore_barrier()

            _writeback(acc, out_ref, r0, last)
            plsc.subcore_barrier()

        def run_core(t_a, t_b, o_a, o_b):
            _fill_rows(zbuf, _ZC, _G, 0.0)
            base = sid * _ROWS_PER_SUB
            pltpu.sync_copy(src2d.at[pl.ds(base, _ROWS_PER_SUB)], srcbuf)
            pltpu.sync_copy(dst2d.at[pl.ds(base, _ROWS_PER_SUB)], dstbuf)
            phase(t_a, o_a)
            phase(t_b, o_b)

        @pl.when(cid == 0)
        def _():
            run_core(t0, t1, o0, o1)

        @pl.when(cid == 1)
        def _():
            run_core(t2, t3, o2, o3)

    return pl.kernel(
        body, out_type=out_type, mesh=mesh, scratch_types=scratch,
        compiler_params=pltpu.CompilerParams(use_tc_tiling_on_sc=False))


def _fill_rows(ref, nrows, ncols, val):
    v = jnp.full((16,), val, jnp.float32)
    def fi(i, _):
        def fj(j, _):
            ref[i, pl.ds(j * 16, 16)] = v
            return 0
        return lax.fori_loop(0, ncols // 16, fj, 0)
    lax.fori_loop(0, nrows, fi, 0)


def _zero_shared(a_ref, src_ref, r0, last):
    def zk(k, _):
        pltpu.sync_copy(src_ref.at[pl.ds(0, _ZC)],
                        a_ref.at[pl.ds(r0 + k * _ZC, _ZC)])
        return 0
    lax.fori_loop(0, _NODE_ROWS // _ZC, zk, 0)
    @pl.when(last)
    def _():
        pltpu.sync_copy(src_ref.at[pl.ds(0, _NODE_EXTRA)],
                        a_ref.at[pl.ds(_NSUB * _NODE_ROWS, _NODE_EXTRA)])


def _writeback(a_ref, out_ref, r0, last):
    pltpu.sync_copy(a_ref.at[pl.ds(r0, _NODE_ROWS)],
                    out_ref.at[pl.ds(r0, _NODE_ROWS)])
    @pl.when(last)
    def _():
        es = pl.ds(_NSUB * _NODE_ROWS, _NODE_EXTRA)
        pltpu.sync_copy(a_ref.at[es], out_ref.at[es])


def _make_sc_count():
    out_type = tuple([jax.ShapeDtypeStruct((N, 16), jnp.float32)] * 2)
    scratch = [
        pltpu.VMEM((_CROWS, _CHUNK), jnp.int32),
        pltpu.VMEM((_CHUNK, 16), jnp.float32),
        pltpu.VMEM((_ZC, 16), jnp.float32),
        pltpu.VMEM_SHARED((_NACC, 16), jnp.float32),
        pltpu.SemaphoreType.DMA,
    ]
    mesh = plsc.VectorSubcoreMesh(core_axis_name="c", subcore_axis_name="s")

    def body(dst2d, o0, o1, dstbuf, ones16, zc16, cacc, semc):
        cid = lax.axis_index("c")
        sid = lax.axis_index("s")
        last = sid == _NSUB - 1
        r0 = sid * _NODE_ROWS

        def drain_c(out_ref):
            pltpu.make_async_copy(out_ref.at[pl.ds(0, _CHUNK)], ones16,
                                  semc).wait()

        def run_core(out_ref):
            _fill_rows(ones16, _CHUNK, 16, 1.0)
            _fill_rows(zc16, _ZC, 16, 0.0)
            base = (cid * _NSUB + sid) * _CROWS
            pltpu.sync_copy(dst2d.at[pl.ds(base, _CROWS)], dstbuf)
            _zero_shared(cacc, zc16, r0, last)
            plsc.subcore_barrier()

            def step(j, _):
                pltpu.async_copy(ones16, cacc.at[dstbuf.at[j]], semc,
                                 add=True)
                @pl.when(j >= 8)
                def _():
                    drain_c(out_ref)
                return 0
            lax.fori_loop(0, _CROWS, step, 0)
            def tail(j, _):
                drain_c(out_ref)
                return 0
            lax.fori_loop(0, 8, tail, 0)
            plsc.subcore_barrier()
            _writeback(cacc, out_ref, r0, last)

        @pl.when(cid == 0)
        def _():
            run_core(o0)

        @pl.when(cid == 1)
        def _():
            run_core(o1)

    return pl.kernel(
        body, out_type=out_type, mesh=mesh, scratch_types=scratch,
        compiler_params=pltpu.CompilerParams(use_tc_tiling_on_sc=False))


_sc_scatter = _make_sc_scatter()
_sc_count = _make_sc_count()


_BN = 1000


def _split_store(t, refs):
    for g, r in enumerate(refs):
        r[...] = t[:, g * _G:(g + 1) * _G]


def _tc_enc_body(x_ref, we_ref, be_ref, w1_ref, b1_ref, *t_refs):
    h = jnp.dot(x_ref[...], we_ref[...], precision=_PREC,
                preferred_element_type=jnp.float32) + be_ref[...]
    t = jnp.maximum(jnp.dot(h, w1_ref[...], precision=_PREC,
                            preferred_element_type=jnp.float32) + b1_ref[...],
                    0.0)
    _split_store(t, t_refs)


def _agg_matmul(s_refs, cnt0_ref, cnt1_ref, w2_ref, b2_ref):
    inv = 1.0 / jnp.maximum(cnt0_ref[:, 0:1] + cnt1_ref[:, 0:1], 1.0)
    h = b2_ref[...]
    for g in range(4):
        h = h + jnp.dot(s_refs[g][...] * inv, w2_ref[g * _G:(g + 1) * _G, :],
                        precision=_PREC, preferred_element_type=jnp.float32)
    return h


def _tc_mid_body(s0, s1, s2, s3, cnt0_ref, cnt1_ref, w2_ref, b2_ref,
                 w1_ref, b1_ref, *t_refs):
    h = _agg_matmul((s0, s1, s2, s3), cnt0_ref, cnt1_ref, w2_ref, b2_ref)
    t = jnp.maximum(jnp.dot(h, w1_ref[...], precision=_PREC,
                            preferred_element_type=jnp.float32) + b1_ref[...],
                    0.0)
    _split_store(t, t_refs)


def _tc_dec_body(s0, s1, s2, s3, cnt0_ref, cnt1_ref, w2_ref, b2_ref,
                 wd_ref, bd_ref, out_ref):
    h = _agg_matmul((s0, s1, s2, s3), cnt0_ref, cnt1_ref, w2_ref, b2_ref)
    out_ref[...] = (jnp.dot(h, wd_ref[...], precision=_PREC,
                            preferred_element_type=jnp.float32) + bd_ref[...])


def _row_spec(w):
    return pl.BlockSpec((_BN, w), lambda i: (i, 0))


def _full_spec(shape):
    return pl.BlockSpec(shape, lambda i: tuple(0 for _ in shape))


_quarter_out = [jax.ShapeDtypeStruct((N, _G), jnp.float32)] * 4
_quarter_specs = [_row_spec(_G)] * 4

_tc_enc = pl.pallas_call(
    _tc_enc_body,
    grid=(N // _BN,),
    in_specs=[_row_spec(D_IN), _full_spec((D_IN, D_H)), _full_spec((1, D_H)),
              _full_spec((D_H, D_H)), _full_spec((1, D_H))],
    out_specs=_quarter_specs,
    out_shape=_quarter_out,
)

_tc_mid = pl.pallas_call(
    _tc_mid_body,
    grid=(N // _BN,),
    in_specs=_quarter_specs + [_row_spec(16), _row_spec(16),
              _full_spec((D_H, D_H)), _full_spec((1, D_H)),
              _full_spec((D_H, D_H)), _full_spec((1, D_H))],
    out_specs=_quarter_specs,
    out_shape=_quarter_out,
)

_tc_dec = pl.pallas_call(
    _tc_dec_body,
    grid=(N // _BN,),
    in_specs=_quarter_specs + [_row_spec(16), _row_spec(16),
              _full_spec((D_H, D_H)), _full_spec((1, D_H)),
              _full_spec((D_H, D_OUT)), _full_spec((1, D_OUT))],
    out_specs=_row_spec(D_OUT),
    out_shape=jax.ShapeDtypeStruct((N, D_OUT), jnp.float32),
)


def kernel(x, edge_index, W_enc, b_enc, W1, b1, W2, b2, W_dec, b_dec):
    pad = _EPAD - E
    src2d = jnp.concatenate(
        [edge_index[0], jnp.zeros((pad,), jnp.int32)]).reshape(_EROWS, _CHUNK)
    dst2d = jnp.concatenate(
        [edge_index[1], jnp.full((pad,), N, jnp.int32)]).reshape(_EROWS, _CHUNK)
    be = b_enc.reshape(1, D_H)
    b1r = b1.reshape(1, D_H)
    b2r = b2.reshape(1, D_H)
    bdr = b_dec.reshape(1, D_OUT)

    cnt0, cnt1 = _sc_count(dst2d)
    t = _tc_enc(x, W_enc, be, W1, b1r)
    s0, s1, s2, s3 = _sc_scatter(t[0], t[1], t[2], t[3], src2d, dst2d)
    u = _tc_mid(s0, s1, s2, s3, cnt0, cnt1, W2, b2r, W1, b1r)
    v0, v1, v2, v3 = _sc_scatter(u[0], u[1], u[2], u[3], src2d, dst2d)
    return _tc_dec(v0, v1, v2, v3, cnt0, cnt1, W2, b2r, W_dec, bdr)


# speedup vs baseline: 3.4958x; 1.0003x over previous
import jax
import jax.numpy as jnp
from jax import lax
from jax.experimental import pallas as pl
from jax.experimental.pallas import tpu as pltpu
from jax.experimental.pallas import tpu_sc as plsc

N = 10000
E = 320000
D_IN = 128
D_H = 256
D_OUT = 128

_NSUB = 16
_CHUNK = 128
_EROWS = 2560
_EPAD = _EROWS * _CHUNK
_ROWS_PER_SUB = _EROWS // _NSUB
_G = 64
_NACC = N + 8
_NODE_ROWS = 624
_NODE_EXTRA = N - _NSUB * _NODE_ROWS
_ZC = 104
_CROWS = _EROWS // 32
_PREC = lax.Precision.HIGHEST


def _make_sc_scatter():
    out_type = tuple([jax.ShapeDtypeStruct((N, _G), jnp.float32)] * 4)

    scratch = [
        pltpu.VMEM((_ROWS_PER_SUB, _CHUNK), jnp.int32),
        pltpu.VMEM((_ROWS_PER_SUB, _CHUNK), jnp.int32),
        pltpu.VMEM((4, _CHUNK, _G), jnp.float32),
        pltpu.VMEM((_ZC, _G), jnp.float32),
        pltpu.VMEM_SHARED((_NACC, _G), jnp.float32),
        pltpu.SemaphoreType.DMA,
        pltpu.SemaphoreType.DMA,
    ]

    mesh = plsc.VectorSubcoreMesh(core_axis_name="c", subcore_axis_name="s")

    def body(t0, t1, t2, t3, src2d, dst2d,
             o0, o1, o2, o3, srcbuf, dstbuf, rb, zbuf, acc, semg, sems):
        cid = lax.axis_index("c")
        sid = lax.axis_index("s")
        last = sid == _NSUB - 1
        r0 = sid * _NODE_ROWS

        def phase(t_ref, out_ref):
            _zero_shared(acc, zbuf, r0, last)
            plsc.subcore_barrier()

            dummy_g = t_ref.at[pl.ds(0, _CHUNK)]

            def drain_g(b):
                pltpu.make_async_copy(dummy_g, rb.at[b], semg).wait()

            def drain_s():
                pltpu.make_async_copy(dummy_g, rb.at[0], sems).wait()

            for p in range(2):
                pltpu.async_copy(t_ref.at[srcbuf.at[p]], rb.at[p], semg)

            def step(j, _):
                b = lax.rem(j, 4)
                drain_g(b)
                pltpu.async_copy(rb.at[b], acc.at[dstbuf.at[j]], sems,
                                 add=True)
                @pl.when(j >= 2)
                def _():
                    drain_s()
                @pl.when(j + 2 < _ROWS_PER_SUB)
                def _():
                    pltpu.async_copy(t_ref.at[srcbuf.at[j + 2]],
                                     rb.at[lax.rem(j + 2, 4)], semg)
                return 0
            lax.fori_loop(0, _ROWS_PER_SUB, step, 0)
            for p in range(2):
                drain_s()
            plsc.subcore_barrier()

            _writeback(acc, out_ref, r0, last)
            plsc.subcore_barrier()

        def run_core(t_a, t_b, o_a, o_b):
            _fill_rows(zbuf, _ZC, _G, 0.0)
            base = sid * _ROWS_PER_SUB
            pltpu.sync_copy(src2d.at[pl.ds(base, _ROWS_PER_SUB)], srcbuf)
            pltpu.sync_copy(dst2d.at[pl.ds(base, _ROWS_PER_SUB)], dstbuf)
            phase(t_a, o_a)
            phase(t_b, o_b)

        @pl.when(cid == 0)
        def _():
            run_core(t0, t1, o0, o1)

        @pl.when(cid == 1)
        def _():
            run_core(t2, t3, o2, o3)

    return pl.kernel(
        body, out_type=out_type, mesh=mesh, scratch_types=scratch,
        compiler_params=pltpu.CompilerParams(use_tc_tiling_on_sc=False))


def _fill_rows(ref, nrows, ncols, val):
    v = jnp.full((16,), val, jnp.float32)
    def fi(i, _):
        def fj(j, _):
            ref[i, pl.ds(j * 16, 16)] = v
            return 0
        return lax.fori_loop(0, ncols // 16, fj, 0)
    lax.fori_loop(0, nrows, fi, 0)


def _zero_shared(a_ref, src_ref, r0, last):
    def zk(k, _):
        pltpu.sync_copy(src_ref.at[pl.ds(0, _ZC)],
                        a_ref.at[pl.ds(r0 + k * _ZC, _ZC)])
        return 0
    lax.fori_loop(0, _NODE_ROWS // _ZC, zk, 0)
    @pl.when(last)
    def _():
        pltpu.sync_copy(src_ref.at[pl.ds(0, _NODE_EXTRA)],
                        a_ref.at[pl.ds(_NSUB * _NODE_ROWS, _NODE_EXTRA)])


def _writeback(a_ref, out_ref, r0, last):
    pltpu.sync_copy(a_ref.at[pl.ds(r0, _NODE_ROWS)],
                    out_ref.at[pl.ds(r0, _NODE_ROWS)])
    @pl.when(last)
    def _():
        es = pl.ds(_NSUB * _NODE_ROWS, _NODE_EXTRA)
        pltpu.sync_copy(a_ref.at[es], out_ref.at[es])


def _make_sc_count():
    out_type = tuple([jax.ShapeDtypeStruct((N, 16), jnp.float32)] * 2)
    scratch = [
        pltpu.VMEM((_CROWS, _CHUNK), jnp.int32),
        pltpu.VMEM((_CHUNK, 16), jnp.float32),
        pltpu.VMEM((_ZC, 16), jnp.float32),
        pltpu.VMEM_SHARED((_NACC, 16), jnp.float32),
        pltpu.SemaphoreType.DMA,
    ]
    mesh = plsc.VectorSubcoreMesh(core_axis_name="c", subcore_axis_name="s")

    def body(dst2d, o0, o1, dstbuf, ones16, zc16, cacc, semc):
        cid = lax.axis_index("c")
        sid = lax.axis_index("s")
        last = sid == _NSUB - 1
        r0 = sid * _NODE_ROWS

        def drain_c(out_ref):
            pltpu.make_async_copy(out_ref.at[pl.ds(0, _CHUNK)], ones16,
                                  semc).wait()

        def run_core(out_ref):
            _fill_rows(ones16, _CHUNK, 16, 1.0)
            _fill_rows(zc16, _ZC, 16, 0.0)
            base = (cid * _NSUB + sid) * _CROWS
            pltpu.sync_copy(dst2d.at[pl.ds(base, _CROWS)], dstbuf)
            _zero_shared(cacc, zc16, r0, last)
            plsc.subcore_barrier()

            def step(j, _):
                pltpu.async_copy(ones16, cacc.at[dstbuf.at[j]], semc,
                                 add=True)
                @pl.when(j >= 8)
                def _():
                    drain_c(out_ref)
                return 0
            lax.fori_loop(0, _CROWS, step, 0)
            def tail(j, _):
                drain_c(out_ref)
                return 0
            lax.fori_loop(0, 8, tail, 0)
            plsc.subcore_barrier()
            _writeback(cacc, out_ref, r0, last)

        @pl.when(cid == 0)
        def _():
            run_core(o0)

        @pl.when(cid == 1)
        def _():
            run_core(o1)

    return pl.kernel(
        body, out_type=out_type, mesh=mesh, scratch_types=scratch,
        compiler_params=pltpu.CompilerParams(use_tc_tiling_on_sc=False))


_sc_scatter = _make_sc_scatter()
_sc_count = _make_sc_count()


_BN = 1000


def _split_store(t, refs):
    for g, r in enumerate(refs):
        r[...] = t[:, g * _G:(g + 1) * _G]


def _tc_enc_body(x_ref, we_ref, be_ref, w1_ref, b1_ref, *t_refs):
    h = jnp.dot(x_ref[...], we_ref[...], precision=_PREC,
                preferred_element_type=jnp.float32) + be_ref[...]
    t = jnp.maximum(jnp.dot(h, w1_ref[...], precision=_PREC,
                            preferred_element_type=jnp.float32) + b1_ref[...],
                    0.0)
    _split_store(t, t_refs)


def _agg_matmul(s_refs, cnt0_ref, cnt1_ref, w2_ref, b2_ref):
    inv = 1.0 / jnp.maximum(cnt0_ref[:, 0:1] + cnt1_ref[:, 0:1], 1.0)
    h = b2_ref[...]
    for g in range(4):
        h = h + jnp.dot(s_refs[g][...] * inv, w2_ref[g * _G:(g + 1) * _G, :],
                        precision=_PREC, preferred_element_type=jnp.float32)
    return h


def _tc_mid_body(s0, s1, s2, s3, cnt0_ref, cnt1_ref, w2_ref, b2_ref,
                 w1_ref, b1_ref, *t_refs):
    h = _agg_matmul((s0, s1, s2, s3), cnt0_ref, cnt1_ref, w2_ref, b2_ref)
    t = jnp.maximum(jnp.dot(h, w1_ref[...], precision=_PREC,
                            preferred_element_type=jnp.float32) + b1_ref[...],
                    0.0)
    _split_store(t, t_refs)


def _tc_dec_body(s0, s1, s2, s3, cnt0_ref, cnt1_ref, w2_ref, b2_ref,
                 wd_ref, bd_ref, out_ref):
    h = _agg_matmul((s0, s1, s2, s3), cnt0_ref, cnt1_ref, w2_ref, b2_ref)
    out_ref[...] = (jnp.dot(h, wd_ref[...], precision=_PREC,
                            preferred_element_type=jnp.float32) + bd_ref[...])


def _row_spec(w):
    return pl.BlockSpec((_BN, w), lambda i: (i, 0))


def _full_spec(shape):
    return pl.BlockSpec(shape, lambda i: tuple(0 for _ in shape))


_quarter_out = [jax.ShapeDtypeStruct((N, _G), jnp.float32)] * 4
_quarter_specs = [_row_spec(_G)] * 4

_tc_enc = pl.pallas_call(
    _tc_enc_body,
    grid=(N // _BN,),
    in_specs=[_row_spec(D_IN), _full_spec((D_IN, D_H)), _full_spec((1, D_H)),
              _full_spec((D_H, D_H)), _full_spec((1, D_H))],
    out_specs=_quarter_specs,
    out_shape=_quarter_out,
)

_tc_mid = pl.pallas_call(
    _tc_mid_body,
    grid=(N // _BN,),
    in_specs=_quarter_specs + [_row_spec(16), _row_spec(16),
              _full_spec((D_H, D_H)), _full_spec((1, D_H)),
              _full_spec((D_H, D_H)), _full_spec((1, D_H))],
    out_specs=_quarter_specs,
    out_shape=_quarter_out,
)

_tc_dec = pl.pallas_call(
    _tc_dec_body,
    grid=(N // _BN,),
    in_specs=_quarter_specs + [_row_spec(16), _row_spec(16),
              _full_spec((D_H, D_H)), _full_spec((1, D_H)),
              _full_spec((D_H, D_OUT)), _full_spec((1, D_OUT))],
    out_specs=_row_spec(D_OUT),
    out_shape=jax.ShapeDtypeStruct((N, D_OUT), jnp.float32),
)


def kernel(x, edge_index, W_enc, b_enc, W1, b1, W2, b2, W_dec, b_dec):
    pad = _EPAD - E
    src2d = jnp.concatenate(
        [edge_index[0], jnp.zeros((pad,), jnp.int32)]).reshape(_EROWS, _CHUNK)
    dst2d = jnp.concatenate(
        [edge_index[1], jnp.full((pad,), N, jnp.int32)]).reshape(_EROWS, _CHUNK)
    be = b_enc.reshape(1, D_H)
    b1r = b1.reshape(1, D_H)
    b2r = b2.reshape(1, D_H)
    bdr = b_dec.reshape(1, D_OUT)

    cnt0, cnt1 = _sc_count(dst2d)
    t = _tc_enc(x, W_enc, be, W1, b1r)
    s0, s1, s2, s3 = _sc_scatter(t[0], t[1], t[2], t[3], src2d, dst2d)
    u = _tc_mid(s0, s1, s2, s3, cnt0, cnt1, W2, b2r, W1, b1r)
    v0, v1, v2, v3 = _sc_scatter(u[0], u[1], u[2], u[3], src2d, dst2d)
    return _tc_dec(v0, v1, v2, v3, cnt0, cnt1, W2, b2r, W_dec, bdr)


# speedup vs baseline: 3.5772x; 1.0233x over previous
import jax
import jax.numpy as jnp
from jax import lax
from jax.experimental import pallas as pl
from jax.experimental.pallas import tpu as pltpu
from jax.experimental.pallas import tpu_sc as plsc

N = 10000
E = 320000
D_IN = 128
D_H = 256
D_OUT = 128

_NSUB = 16
_CHUNK = 128
_EROWS = 2560
_EPAD = _EROWS * _CHUNK
_ROWS_PER_SUB = _EROWS // _NSUB
_G = 64
_NACC = N + 8
_NODE_ROWS = 624
_NODE_EXTRA = N - _NSUB * _NODE_ROWS
_ZC = 104
_SLAB = 32
_NSLAB = _ROWS_PER_SUB // _SLAB
_CROWS = _EROWS // 32
_PREC = lax.Precision.HIGHEST


def _make_sc_scatter():
    out_type = tuple([jax.ShapeDtypeStruct((N, _G), jnp.float32)] * 4)

    scratch = [
        pltpu.VMEM((2, _SLAB, _CHUNK), jnp.int32),
        pltpu.VMEM((2, _SLAB, _CHUNK), jnp.int32),
        pltpu.VMEM((8, _CHUNK, _G), jnp.float32),
        pltpu.VMEM((_ZC, _G), jnp.float32),
        pltpu.VMEM_SHARED((_NACC, _G), jnp.float32),
        pltpu.SemaphoreType.DMA,
        pltpu.SemaphoreType.DMA,
        pltpu.SemaphoreType.DMA,
    ]

    mesh = plsc.VectorSubcoreMesh(core_axis_name="c", subcore_axis_name="s")

    def body(t0, t1, t2, t3, src2d, dst2d,
             o0, o1, o2, o3, srcsl, dstsl, rb, zbuf, acc, semg, sems, semi):
        cid = lax.axis_index("c")
        sid = lax.axis_index("s")
        last = sid == _NSUB - 1
        r0 = sid * _NODE_ROWS
        base = sid * _ROWS_PER_SUB

        def phase(t_ref, out_ref):
            _zero_shared(acc, zbuf, r0, last)
            plsc.subcore_barrier()

            dummy_g = t_ref.at[pl.ds(0, _CHUNK)]
            dummy_i = src2d.at[pl.ds(0, _SLAB)]

            def drain_g():
                pltpu.make_async_copy(dummy_g, rb.at[0], semg).wait()

            def drain_s():
                pltpu.make_async_copy(dummy_g, rb.at[0], sems).wait()

            def drain_i():
                pltpu.make_async_copy(dummy_i, srcsl.at[0], semi).wait()

            pltpu.sync_copy(src2d.at[pl.ds(base, _SLAB)], srcsl.at[0])
            pltpu.sync_copy(dst2d.at[pl.ds(base, _SLAB)], dstsl.at[0])
            for p in range(4):
                pltpu.async_copy(t_ref.at[srcsl.at[0, p]], rb.at[p], semg)

            def slab_loop(k, _):
                kb = lax.rem(k, 2)
                nb = lax.rem(k + 1, 2)

                @pl.when(k + 1 < _NSLAB)
                def _():
                    off = base + (k + 1) * _SLAB
                    pltpu.async_copy(src2d.at[pl.ds(off, _SLAB)],
                                     srcsl.at[nb], semi)
                    pltpu.async_copy(dst2d.at[pl.ds(off, _SLAB)],
                                     dstsl.at[nb], semi)

                def step(i, _):
                    j = k * _SLAB + i
                    b = lax.rem(j, 8)
                    drain_g()
                    pltpu.async_copy(rb.at[b], acc.at[dstsl.at[kb, i]],
                                     sems, add=True)
                    @pl.when(j >= 4)
                    def _():
                        drain_s()
                    @pl.when(jnp.logical_and(i == _SLAB - 4,
                                             k + 1 < _NSLAB))
                    def _():
                        drain_i()
                        drain_i()
                    @pl.when(j + 4 < _ROWS_PER_SUB)
                    def _():
                        bn = rb.at[lax.rem(j + 4, 8)]
                        @pl.when(i < _SLAB - 4)
                        def _():
                            pltpu.async_copy(t_ref.at[srcsl.at[kb, i + 4]],
                                             bn, semg)
                        @pl.when(i >= _SLAB - 4)
                        def _():
                            pltpu.async_copy(
                                t_ref.at[srcsl.at[nb, i + 4 - _SLAB]],
                                bn, semg)
                    return 0
                lax.fori_loop(0, _SLAB, step, 0)
                return 0
            lax.fori_loop(0, _NSLAB, slab_loop, 0)
            for p in range(4):
                drain_s()
            plsc.subcore_barrier()

            _writeback(acc, out_ref, r0, last)
            plsc.subcore_barrier()

        def run_core(t_a, t_b, o_a, o_b):
            _fill_rows(zbuf, _ZC, _G, 0.0)
            phase(t_a, o_a)
            phase(t_b, o_b)

        @pl.when(cid == 0)
        def _():
            run_core(t0, t1, o0, o1)

        @pl.when(cid == 1)
        def _():
            run_core(t2, t3, o2, o3)

    return pl.kernel(
        body, out_type=out_type, mesh=mesh, scratch_types=scratch,
        compiler_params=pltpu.CompilerParams(use_tc_tiling_on_sc=False))


def _fill_rows(ref, nrows, ncols, val):
    v = jnp.full((16,), val, jnp.float32)
    def fi(i, _):
        def fj(j, _):
            ref[i, pl.ds(j * 16, 16)] = v
            return 0
        return lax.fori_loop(0, ncols // 16, fj, 0)
    lax.fori_loop(0, nrows, fi, 0)


def _zero_shared(a_ref, src_ref, r0, last):
    def zk(k, _):
        pltpu.sync_copy(src_ref.at[pl.ds(0, _ZC)],
                        a_ref.at[pl.ds(r0 + k * _ZC, _ZC)])
        return 0
    lax.fori_loop(0, _NODE_ROWS // _ZC, zk, 0)
    @pl.when(last)
    def _():
        pltpu.sync_copy(src_ref.at[pl.ds(0, _NODE_EXTRA)],
                        a_ref.at[pl.ds(_NSUB * _NODE_ROWS, _NODE_EXTRA)])


def _writeback(a_ref, out_ref, r0, last):
    pltpu.sync_copy(a_ref.at[pl.ds(r0, _NODE_ROWS)],
                    out_ref.at[pl.ds(r0, _NODE_ROWS)])
    @pl.when(last)
    def _():
        es = pl.ds(_NSUB * _NODE_ROWS, _NODE_EXTRA)
        pltpu.sync_copy(a_ref.at[es], out_ref.at[es])


def _make_sc_count():
    out_type = tuple([jax.ShapeDtypeStruct((N, 16), jnp.float32)] * 2)
    scratch = [
        pltpu.VMEM((_CROWS, _CHUNK), jnp.int32),
        pltpu.VMEM((_CHUNK, 16), jnp.float32),
        pltpu.VMEM((_ZC, 16), jnp.float32),
        pltpu.VMEM_SHARED((_NACC, 16), jnp.float32),
        pltpu.SemaphoreType.DMA,
    ]
    mesh = plsc.VectorSubcoreMesh(core_axis_name="c", subcore_axis_name="s")

    def body(dst2d, o0, o1, dstbuf, ones16, zc16, cacc, semc):
        cid = lax.axis_index("c")
        sid = lax.axis_index("s")
        last = sid == _NSUB - 1
        r0 = sid * _NODE_ROWS

        def drain_c(out_ref):
            pltpu.make_async_copy(out_ref.at[pl.ds(0, _CHUNK)], ones16,
                                  semc).wait()

        def run_core(out_ref):
            _fill_rows(ones16, _CHUNK, 16, 1.0)
            _fill_rows(zc16, _ZC, 16, 0.0)
            base = (cid * _NSUB + sid) * _CROWS
            pltpu.sync_copy(dst2d.at[pl.ds(base, _CROWS)], dstbuf)
            _zero_shared(cacc, zc16, r0, last)
            plsc.subcore_barrier()

            def step(j, _):
                pltpu.async_copy(ones16, cacc.at[dstbuf.at[j]], semc,
                                 add=True)
                @pl.when(j >= 8)
                def _():
                    drain_c(out_ref)
                return 0
            lax.fori_loop(0, _CROWS, step, 0)
            def tail(j, _):
                drain_c(out_ref)
                return 0
            lax.fori_loop(0, 8, tail, 0)
            plsc.subcore_barrier()
            _writeback(cacc, out_ref, r0, last)

        @pl.when(cid == 0)
        def _():
            run_core(o0)

        @pl.when(cid == 1)
        def _():
            run_core(o1)

    return pl.kernel(
        body, out_type=out_type, mesh=mesh, scratch_types=scratch,
        compiler_params=pltpu.CompilerParams(use_tc_tiling_on_sc=False))


_sc_scatter = _make_sc_scatter()
_sc_count = _make_sc_count()


_BN = 1000


def _split_store(t, refs):
    for g, r in enumerate(refs):
        r[...] = t[:, g * _G:(g + 1) * _G]


def _tc_enc_body(x_ref, we_ref, be_ref, w1_ref, b1_ref, *t_refs):
    h = jnp.dot(x_ref[...], we_ref[...], precision=_PREC,
                preferred_element_type=jnp.float32) + be_ref[...]
    t = jnp.maximum(jnp.dot(h, w1_ref[...], precision=_PREC,
                            preferred_element_type=jnp.float32) + b1_ref[...],
                    0.0)
    _split_store(t, t_refs)


def _agg_matmul(s_refs, cnt0_ref, cnt1_ref, w2_ref, b2_ref):
    inv = 1.0 / jnp.maximum(cnt0_ref[:, 0:1] + cnt1_ref[:, 0:1], 1.0)
    h = b2_ref[...]
    for g in range(4):
        h = h + jnp.dot(s_refs[g][...] * inv, w2_ref[g * _G:(g + 1) * _G, :],
                        precision=_PREC, preferred_element_type=jnp.float32)
    return h


def _tc_mid_body(s0, s1, s2, s3, cnt0_ref, cnt1_ref, w2_ref, b2_ref,
                 w1_ref, b1_ref, *t_refs):
    h = _agg_matmul((s0, s1, s2, s3), cnt0_ref, cnt1_ref, w2_ref, b2_ref)
    t = jnp.maximum(jnp.dot(h, w1_ref[...], precision=_PREC,
                            preferred_element_type=jnp.float32) + b1_ref[...],
                    0.0)
    _split_store(t, t_refs)


def _tc_dec_body(s0, s1, s2, s3, cnt0_ref, cnt1_ref, w2_ref, b2_ref,
                 wd_ref, bd_ref, out_ref):
    h = _agg_matmul((s0, s1, s2, s3), cnt0_ref, cnt1_ref, w2_ref, b2_ref)
    out_ref[...] = (jnp.dot(h, wd_ref[...], precision=_PREC,
                            preferred_element_type=jnp.float32) + bd_ref[...])


def _row_spec(w):
    return pl.BlockSpec((_BN, w), lambda i: (i, 0))


def _full_spec(shape):
    return pl.BlockSpec(shape, lambda i: tuple(0 for _ in shape))


_quarter_out = [jax.ShapeDtypeStruct((N, _G), jnp.float32)] * 4
_quarter_specs = [_row_spec(_G)] * 4

_tc_enc = pl.pallas_call(
    _tc_enc_body,
    grid=(N // _BN,),
    in_specs=[_row_spec(D_IN), _full_spec((D_IN, D_H)), _full_spec((1, D_H)),
              _full_spec((D_H, D_H)), _full_spec((1, D_H))],
    out_specs=_quarter_specs,
    out_shape=_quarter_out,
)

_tc_mid = pl.pallas_call(
    _tc_mid_body,
    grid=(N // _BN,),
    in_specs=_quarter_specs + [_row_spec(16), _row_spec(16),
              _full_spec((D_H, D_H)), _full_spec((1, D_H)),
              _full_spec((D_H, D_H)), _full_spec((1, D_H))],
    out_specs=_quarter_specs,
    out_shape=_quarter_out,
)

_tc_dec = pl.pallas_call(
    _tc_dec_body,
    grid=(N // _BN,),
    in_specs=_quarter_specs + [_row_spec(16), _row_spec(16),
              _full_spec((D_H, D_H)), _full_spec((1, D_H)),
              _full_spec((D_H, D_OUT)), _full_spec((1, D_OUT))],
    out_specs=_row_spec(D_OUT),
    out_shape=jax.ShapeDtypeStruct((N, D_OUT), jnp.float32),
)


def kernel(x, edge_index, W_enc, b_enc, W1, b1, W2, b2, W_dec, b_dec):
    pad = _EPAD - E
    src2d = jnp.concatenate(
        [edge_index[0], jnp.zeros((pad,), jnp.int32)]).reshape(_EROWS, _CHUNK)
    dst2d = jnp.concatenate(
        [edge_index[1], jnp.full((pad,), N, jnp.int32)]).reshape(_EROWS, _CHUNK)
    be = b_enc.reshape(1, D_H)
    b1r = b1.reshape(1, D_H)
    b2r = b2.reshape(1, D_H)
    bdr = b_dec.reshape(1, D_OUT)

    cnt0, cnt1 = _sc_count(dst2d)
    t = _tc_enc(x, W_enc, be, W1, b1r)
    s0, s1, s2, s3 = _sc_scatter(t[0], t[1], t[2], t[3], src2d, dst2d)
    u = _tc_mid(s0, s1, s2, s3, cnt0, cnt1, W2, b2r, W1, b1r)
    v0, v1, v2, v3 = _sc_scatter(u[0], u[1], u[2], u[3], src2d, dst2d)
    return _tc_dec(v0, v1, v2, v3, cnt0, cnt1, W2, b2r, W_dec, bdr)


# speedup vs baseline: 4.1535x; 1.1611x over previous
import jax
import jax.numpy as jnp
from jax import lax
from jax.experimental import pallas as pl
from jax.experimental.pallas import tpu as pltpu
from jax.experimental.pallas import tpu_sc as plsc

N = 10000
E = 320000
D_IN = 128
D_H = 256
D_OUT = 128

_NSUB = 16
_H = 128
_C2 = 64
_EPAD = 327680
_EROWS2 = _EPAD // _C2
_RPS2 = _EROWS2 // _NSUB
_SLAB = 32
_NSLAB = _RPS2 // _SLAB
_NACC = N + 8
_NODE_ROWS = 624
_NODE_EXTRA = N - _NSUB * _NODE_ROWS
_ZC = 52
_CCHUNK = 128
_CEROWS = _EPAD // _CCHUNK
_CROWS = _CEROWS // 32
_PREC = lax.Precision.HIGHEST


def _make_sc_scatter():
    out_type = tuple([jax.ShapeDtypeStruct((N, _H), jnp.float32)] * 2)

    scratch = [
        pltpu.VMEM((2, _SLAB, _C2), jnp.int32),
        pltpu.VMEM((2, _SLAB, _C2), jnp.int32),
        pltpu.VMEM((4, _C2, _H), jnp.float32),
        pltpu.VMEM((_ZC, _H), jnp.float32),
        pltpu.VMEM_SHARED((_NACC, _H), jnp.float32),
        pltpu.SemaphoreType.DMA,
        pltpu.SemaphoreType.DMA,
        pltpu.SemaphoreType.DMA,
    ]

    mesh = plsc.VectorSubcoreMesh(core_axis_name="c", subcore_axis_name="s")

    def body(tA, tB, src2d, dst2d,
             oA, oB, srcsl, dstsl, rb, zbuf, acc, semg, sems, semi):
        cid = lax.axis_index("c")
        sid = lax.axis_index("s")
        last = sid == _NSUB - 1
        r0 = sid * _NODE_ROWS
        base = sid * _RPS2

        def run_core(t_ref, out_ref):
            _fill_rows(zbuf, _ZC, _H, 0.0)
            _zero_shared(acc, zbuf, r0, last)
            plsc.subcore_barrier()

            dummy_g = t_ref.at[pl.ds(0, _C2)]
            dummy_i = src2d.at[pl.ds(0, _SLAB)]

            def drain_g():
                pltpu.make_async_copy(dummy_g, rb.at[0], semg).wait()

            def drain_s():
                pltpu.make_async_copy(dummy_g, rb.at[0], sems).wait()

            def drain_i():
                pltpu.make_async_copy(dummy_i, srcsl.at[0], semi).wait()

            pltpu.sync_copy(src2d.at[pl.ds(base, _SLAB)], srcsl.at[0])
            pltpu.sync_copy(dst2d.at[pl.ds(base, _SLAB)], dstsl.at[0])
            for p in range(2):
                pltpu.async_copy(t_ref.at[srcsl.at[0, p]], rb.at[p], semg)

            def slab_loop(k, _):
                kb = lax.rem(k, 2)
                nb = lax.rem(k + 1, 2)

                @pl.when(k + 1 < _NSLAB)
                def _():
                    off = base + (k + 1) * _SLAB
                    pltpu.async_copy(src2d.at[pl.ds(off, _SLAB)],
                                     srcsl.at[nb], semi)
                    pltpu.async_copy(dst2d.at[pl.ds(off, _SLAB)],
                                     dstsl.at[nb], semi)

                def step(i, _):
                    j = k * _SLAB + i
                    b = lax.rem(j, 4)
                    drain_g()
                    pltpu.async_copy(rb.at[b], acc.at[dstsl.at[kb, i]],
                                     sems, add=True)
                    @pl.when(j >= 2)
                    def _():
                        drain_s()
                    @pl.when(jnp.logical_and(i == _SLAB - 2,
                                             k + 1 < _NSLAB))
                    def _():
                        drain_i()
                        drain_i()
                    @pl.when(j + 2 < _RPS2)
                    def _():
                        bn = rb.at[lax.rem(j + 2, 4)]
                        @pl.when(i < _SLAB - 2)
                        def _():
                            pltpu.async_copy(t_ref.at[srcsl.at[kb, i + 2]],
                                             bn, semg)
                        @pl.when(i >= _SLAB - 2)
                        def _():
                            pltpu.async_copy(
                                t_ref.at[srcsl.at[nb, i + 2 - _SLAB]],
                                bn, semg)
                    return 0
                lax.fori_loop(0, _SLAB, step, 0)
                return 0
            lax.fori_loop(0, _NSLAB, slab_loop, 0)
            for p in range(2):
                drain_s()
            plsc.subcore_barrier()

            _writeback(acc, out_ref, r0, last)
            plsc.subcore_barrier()

        @pl.when(cid == 0)
        def _():
            run_core(tA, oA)

        @pl.when(cid == 1)
        def _():
            run_core(tB, oB)

    return pl.kernel(
        body, out_type=out_type, mesh=mesh, scratch_types=scratch,
        compiler_params=pltpu.CompilerParams(use_tc_tiling_on_sc=False))


def _fill_rows(ref, nrows, ncols, val):
    v = jnp.full((16,), val, jnp.float32)
    def fi(i, _):
        def fj(j, _):
            ref[i, pl.ds(j * 16, 16)] = v
            return 0
        return lax.fori_loop(0, ncols // 16, fj, 0)
    lax.fori_loop(0, nrows, fi, 0)


def _zero_shared(a_ref, src_ref, r0, last):
    def zk(k, _):
        pltpu.sync_copy(src_ref.at[pl.ds(0, _ZC)],
                        a_ref.at[pl.ds(r0 + k * _ZC, _ZC)])
        return 0
    lax.fori_loop(0, _NODE_ROWS // _ZC, zk, 0)
    @pl.when(last)
    def _():
        pltpu.sync_copy(src_ref.at[pl.ds(0, _NODE_EXTRA)],
                        a_ref.at[pl.ds(_NSUB * _NODE_ROWS, _NODE_EXTRA)])


def _writeback(a_ref, out_ref, r0, last):
    pltpu.sync_copy(a_ref.at[pl.ds(r0, _NODE_ROWS)],
                    out_ref.at[pl.ds(r0, _NODE_ROWS)])
    @pl.when(last)
    def _():
        es = pl.ds(_NSUB * _NODE_ROWS, _NODE_EXTRA)
        pltpu.sync_copy(a_ref.at[es], out_ref.at[es])


def _make_sc_count():
    out_type = tuple([jax.ShapeDtypeStruct((N, 16), jnp.float32)] * 2)
    scratch = [
        pltpu.VMEM((_CROWS, _CCHUNK), jnp.int32),
        pltpu.VMEM((_CCHUNK, 16), jnp.float32),
        pltpu.VMEM((_ZC, 16), jnp.float32),
        pltpu.VMEM_SHARED((_NACC, 16), jnp.float32),
        pltpu.SemaphoreType.DMA,
    ]
    mesh = plsc.VectorSubcoreMesh(core_axis_name="c", subcore_axis_name="s")

    def body(dst2d, o0, o1, dstbuf, ones16, zc16, cacc, semc):
        cid = lax.axis_index("c")
        sid = lax.axis_index("s")
        last = sid == _NSUB - 1
        r0 = sid * _NODE_ROWS

        def drain_c(out_ref):
            pltpu.make_async_copy(out_ref.at[pl.ds(0, _CCHUNK)], ones16,
                                  semc).wait()

        def run_core(out_ref):
            _fill_rows(ones16, _CCHUNK, 16, 1.0)
            _fill_rows(zc16, _ZC, 16, 0.0)
            base = (cid * _NSUB + sid) * _CROWS
            pltpu.sync_copy(dst2d.at[pl.ds(base, _CROWS)], dstbuf)
            _zero_shared(cacc, zc16, r0, last)
            plsc.subcore_barrier()

            def step(j, _):
                pltpu.async_copy(ones16, cacc.at[dstbuf.at[j]], semc,
                                 add=True)
                @pl.when(j >= 8)
                def _():
                    drain_c(out_ref)
                return 0
            lax.fori_loop(0, _CROWS, step, 0)
            def tail(j, _):
                drain_c(out_ref)
                return 0
            lax.fori_loop(0, 8, tail, 0)
            plsc.subcore_barrier()
            _writeback(cacc, out_ref, r0, last)

        @pl.when(cid == 0)
        def _():
            run_core(o0)

        @pl.when(cid == 1)
        def _():
            run_core(o1)

    return pl.kernel(
        body, out_type=out_type, mesh=mesh, scratch_types=scratch,
        compiler_params=pltpu.CompilerParams(use_tc_tiling_on_sc=False))


_sc_scatter = _make_sc_scatter()
_sc_count = _make_sc_count()


_BN = 1000


def _split_store(t, refs):
    for g, r in enumerate(refs):
        r[...] = t[:, g * _H:(g + 1) * _H]


def _tc_enc_body(x_ref, we_ref, be_ref, w1_ref, b1_ref, *t_refs):
    h = jnp.dot(x_ref[...], we_ref[...], precision=_PREC,
                preferred_element_type=jnp.float32) + be_ref[...]
    t = jnp.maximum(jnp.dot(h, w1_ref[...], precision=_PREC,
                            preferred_element_type=jnp.float32) + b1_ref[...],
                    0.0)
    _split_store(t, t_refs)


def _agg_matmul(s_refs, cnt0_ref, cnt1_ref, w2_ref, b2_ref):
    inv = 1.0 / jnp.maximum(cnt0_ref[:, 0:1] + cnt1_ref[:, 0:1], 1.0)
    h = b2_ref[...]
    for g in range(2):
        h = h + jnp.dot(s_refs[g][...] * inv, w2_ref[g * _H:(g + 1) * _H, :],
                        precision=_PREC, preferred_element_type=jnp.float32)
    return h


def _tc_mid_body(s0, s1, cnt0_ref, cnt1_ref, w2_ref, b2_ref,
                 w1_ref, b1_ref, *t_refs):
    h = _agg_matmul((s0, s1), cnt0_ref, cnt1_ref, w2_ref, b2_ref)
    t = jnp.maximum(jnp.dot(h, w1_ref[...], precision=_PREC,
                            preferred_element_type=jnp.float32) + b1_ref[...],
                    0.0)
    _split_store(t, t_refs)


def _tc_dec_body(s0, s1, cnt0_ref, cnt1_ref, w2_ref, b2_ref,
                 wd_ref, bd_ref, out_ref):
    h = _agg_matmul((s0, s1), cnt0_ref, cnt1_ref, w2_ref, b2_ref)
    out_ref[...] = (jnp.dot(h, wd_ref[...], precision=_PREC,
                            preferred_element_type=jnp.float32) + bd_ref[...])


def _row_spec(w):
    return pl.BlockSpec((_BN, w), lambda i: (i, 0))


def _full_spec(shape):
    return pl.BlockSpec(shape, lambda i: tuple(0 for _ in shape))


_half_out = [jax.ShapeDtypeStruct((N, _H), jnp.float32)] * 2
_half_specs = [_row_spec(_H)] * 2

_tc_enc = pl.pallas_call(
    _tc_enc_body,
    grid=(N // _BN,),
    in_specs=[_row_spec(D_IN), _full_spec((D_IN, D_H)), _full_spec((1, D_H)),
              _full_spec((D_H, D_H)), _full_spec((1, D_H))],
    out_specs=_half_specs,
    out_shape=_half_out,
)

_tc_mid = pl.pallas_call(
    _tc_mid_body,
    grid=(N // _BN,),
    in_specs=_half_specs + [_row_spec(16), _row_spec(16),
              _full_spec((D_H, D_H)), _full_spec((1, D_H)),
              _full_spec((D_H, D_H)), _full_spec((1, D_H))],
    out_specs=_half_specs,
    out_shape=_half_out,
)

_tc_dec = pl.pallas_call(
    _tc_dec_body,
    grid=(N // _BN,),
    in_specs=_half_specs + [_row_spec(16), _row_spec(16),
              _full_spec((D_H, D_H)), _full_spec((1, D_H)),
              _full_spec((D_H, D_OUT)), _full_spec((1, D_OUT))],
    out_specs=_row_spec(D_OUT),
    out_shape=jax.ShapeDtypeStruct((N, D_OUT), jnp.float32),
)


def kernel(x, edge_index, W_enc, b_enc, W1, b1, W2, b2, W_dec, b_dec):
    pad = _EPAD - E
    src_p = jnp.concatenate([edge_index[0], jnp.zeros((pad,), jnp.int32)])
    dst_p = jnp.concatenate([edge_index[1], jnp.full((pad,), N, jnp.int32)])
    src2e = src_p.reshape(_EROWS2, _C2)
    dst2e = dst_p.reshape(_EROWS2, _C2)
    dst2c = dst_p.reshape(_CEROWS, _CCHUNK)
    be = b_enc.reshape(1, D_H)
    b1r = b1.reshape(1, D_H)
    b2r = b2.reshape(1, D_H)
    bdr = b_dec.reshape(1, D_OUT)

    cnt0, cnt1 = _sc_count(dst2c)
    t = _tc_enc(x, W_enc, be, W1, b1r)
    s0, s1 = _sc_scatter(t[0], t[1], src2e, dst2e)
    u = _tc_mid(s0, s1, cnt0, cnt1, W2, b2r, W1, b1r)
    v0, v1 = _sc_scatter(u[0], u[1], src2e, dst2e)
    return _tc_dec(v0, v1, cnt0, cnt1, W2, b2r, W_dec, bdr)


# speedup vs baseline: 4.2412x; 1.0211x over previous
import jax
import jax.numpy as jnp
from jax import lax
from jax.experimental import pallas as pl
from jax.experimental.pallas import tpu as pltpu
from jax.experimental.pallas import tpu_sc as plsc

N = 10000
E = 320000
D_IN = 128
D_H = 256
D_OUT = 128

_NSUB = 16
_H = 128
_C2 = 64
_EPAD = 327680
_EROWS2 = _EPAD // _C2
_RPS2 = _EROWS2 // _NSUB
_SLAB = 32
_NSLAB = _RPS2 // _SLAB
_NACC = N + 8
_NODE_ROWS = 624
_NODE_EXTRA = N - _NSUB * _NODE_ROWS
_ZC = 52
_CCHUNK = 128
_CEROWS = _EPAD // _CCHUNK
_CROWS = _CEROWS // 32
_PREC = lax.Precision.HIGHEST


def _make_sc_scatter():
    out_type = tuple([jax.ShapeDtypeStruct((N, _H), jnp.float32)] * 2)

    scratch = [
        pltpu.VMEM((2, _SLAB, _C2), jnp.int32),
        pltpu.VMEM((2, _SLAB, _C2), jnp.int32),
        pltpu.VMEM((5, _C2, _H), jnp.float32),
        pltpu.VMEM_SHARED((_NACC, _H), jnp.float32),
        pltpu.SemaphoreType.DMA,
        pltpu.SemaphoreType.DMA,
        pltpu.SemaphoreType.DMA,
    ]

    mesh = plsc.VectorSubcoreMesh(core_axis_name="c", subcore_axis_name="s")

    def body(tA, tB, src2d, dst2d,
             oA, oB, srcsl, dstsl, rb, acc, semg, sems, semi):
        cid = lax.axis_index("c")
        sid = lax.axis_index("s")
        last = sid == _NSUB - 1
        r0 = sid * _NODE_ROWS
        base = sid * _RPS2

        def run_core(t_ref, out_ref):
            _fill_rows(rb.at[0], _ZC, _H, 0.0)
            _zero_shared(acc, rb.at[0], r0, last)
            plsc.subcore_barrier()

            dummy_g = t_ref.at[pl.ds(0, _C2)]
            dummy_i = src2d.at[pl.ds(0, _SLAB)]

            def drain_g():
                pltpu.make_async_copy(dummy_g, rb.at[0], semg).wait()

            def drain_s():
                pltpu.make_async_copy(dummy_g, rb.at[0], sems).wait()

            def drain_i():
                pltpu.make_async_copy(dummy_i, srcsl.at[0], semi).wait()

            pltpu.sync_copy(src2d.at[pl.ds(base, _SLAB)], srcsl.at[0])
            pltpu.sync_copy(dst2d.at[pl.ds(base, _SLAB)], dstsl.at[0])
            for p in range(3):
                pltpu.async_copy(t_ref.at[srcsl.at[0, p]], rb.at[p], semg)

            def slab_loop(k, _):
                kb = lax.rem(k, 2)
                nb = lax.rem(k + 1, 2)

                @pl.when(k + 1 < _NSLAB)
                def _():
                    off = base + (k + 1) * _SLAB
                    pltpu.async_copy(src2d.at[pl.ds(off, _SLAB)],
                                     srcsl.at[nb], semi)
                    pltpu.async_copy(dst2d.at[pl.ds(off, _SLAB)],
                                     dstsl.at[nb], semi)

                def step(i, _):
                    j = k * _SLAB + i
                    b = lax.rem(j, 5)
                    drain_g()
                    pltpu.async_copy(rb.at[b], acc.at[dstsl.at[kb, i]],
                                     sems, add=True)
                    @pl.when(j >= 2)
                    def _():
                        drain_s()
                    @pl.when(jnp.logical_and(i == _SLAB - 3,
                                             k + 1 < _NSLAB))
                    def _():
                        drain_i()
                        drain_i()
                    @pl.when(j + 3 < _RPS2)
                    def _():
                        bn = rb.at[lax.rem(j + 3, 5)]
                        @pl.when(i < _SLAB - 3)
                        def _():
                            pltpu.async_copy(t_ref.at[srcsl.at[kb, i + 3]],
                                             bn, semg)
                        @pl.when(i >= _SLAB - 3)
                        def _():
                            pltpu.async_copy(
                                t_ref.at[srcsl.at[nb, i + 3 - _SLAB]],
                                bn, semg)
                    return 0
                lax.fori_loop(0, _SLAB, step, 0)
                return 0
            lax.fori_loop(0, _NSLAB, slab_loop, 0)
            for p in range(2):
                drain_s()
            plsc.subcore_barrier()

            _writeback(acc, out_ref, r0, last)
            plsc.subcore_barrier()

        @pl.when(cid == 0)
        def _():
            run_core(tA, oA)

        @pl.when(cid == 1)
        def _():
            run_core(tB, oB)

    return pl.kernel(
        body, out_type=out_type, mesh=mesh, scratch_types=scratch,
        compiler_params=pltpu.CompilerParams(use_tc_tiling_on_sc=False))


def _fill_rows(ref, nrows, ncols, val):
    v = jnp.full((16,), val, jnp.float32)
    def fi(i, _):
        def fj(j, _):
            ref[i, pl.ds(j * 16, 16)] = v
            return 0
        return lax.fori_loop(0, ncols // 16, fj, 0)
    lax.fori_loop(0, nrows, fi, 0)


def _zero_shared(a_ref, src_ref, r0, last):
    def zk(k, _):
        pltpu.sync_copy(src_ref.at[pl.ds(0, _ZC)],
                        a_ref.at[pl.ds(r0 + k * _ZC, _ZC)])
        return 0
    lax.fori_loop(0, _NODE_ROWS // _ZC, zk, 0)
    @pl.when(last)
    def _():
        pltpu.sync_copy(src_ref.at[pl.ds(0, _NODE_EXTRA)],
                        a_ref.at[pl.ds(_NSUB * _NODE_ROWS, _NODE_EXTRA)])


def _writeback(a_ref, out_ref, r0, last):
    pltpu.sync_copy(a_ref.at[pl.ds(r0, _NODE_ROWS)],
                    out_ref.at[pl.ds(r0, _NODE_ROWS)])
    @pl.when(last)
    def _():
        es = pl.ds(_NSUB * _NODE_ROWS, _NODE_EXTRA)
        pltpu.sync_copy(a_ref.at[es], out_ref.at[es])


def _make_sc_count():
    out_type = tuple([jax.ShapeDtypeStruct((N, 16), jnp.float32)] * 2)
    scratch = [
        pltpu.VMEM((_CROWS, _CCHUNK), jnp.int32),
        pltpu.VMEM((_CCHUNK, 16), jnp.float32),
        pltpu.VMEM((_ZC, 16), jnp.float32),
        pltpu.VMEM_SHARED((_NACC, 16), jnp.float32),
        pltpu.SemaphoreType.DMA,
    ]
    mesh = plsc.VectorSubcoreMesh(core_axis_name="c", subcore_axis_name="s")

    def body(dst2d, o0, o1, dstbuf, ones16, zc16, cacc, semc):
        cid = lax.axis_index("c")
        sid = lax.axis_index("s")
        last = sid == _NSUB - 1
        r0 = sid * _NODE_ROWS

        def drain_c(out_ref):
            pltpu.make_async_copy(out_ref.at[pl.ds(0, _CCHUNK)], ones16,
                                  semc).wait()

        def run_core(out_ref):
            _fill_rows(ones16, _CCHUNK, 16, 1.0)
            _fill_rows(zc16, _ZC, 16, 0.0)
            base = (cid * _NSUB + sid) * _CROWS
            pltpu.sync_copy(dst2d.at[pl.ds(base, _CROWS)], dstbuf)
            _zero_shared(cacc, zc16, r0, last)
            plsc.subcore_barrier()

            def step(j, _):
                pltpu.async_copy(ones16, cacc.at[dstbuf.at[j]], semc,
                                 add=True)
                @pl.when(j >= 8)
                def _():
                    drain_c(out_ref)
                return 0
            lax.fori_loop(0, _CROWS, step, 0)
            def tail(j, _):
                drain_c(out_ref)
                return 0
            lax.fori_loop(0, 8, tail, 0)
            plsc.subcore_barrier()
            _writeback(cacc, out_ref, r0, last)

        @pl.when(cid == 0)
        def _():
            run_core(o0)

        @pl.when(cid == 1)
        def _():
            run_core(o1)

    return pl.kernel(
        body, out_type=out_type, mesh=mesh, scratch_types=scratch,
        compiler_params=pltpu.CompilerParams(use_tc_tiling_on_sc=False))


_sc_scatter = _make_sc_scatter()
_sc_count = _make_sc_count()


_BN = 1000


def _split_store(t, refs):
    for g, r in enumerate(refs):
        r[...] = t[:, g * _H:(g + 1) * _H]


def _tc_enc_body(x_ref, we_ref, be_ref, w1_ref, b1_ref, *t_refs):
    h = jnp.dot(x_ref[...], we_ref[...], precision=_PREC,
                preferred_element_type=jnp.float32) + be_ref[...]
    t = jnp.maximum(jnp.dot(h, w1_ref[...], precision=_PREC,
                            preferred_element_type=jnp.float32) + b1_ref[...],
                    0.0)
    _split_store(t, t_refs)


def _agg_matmul(s_refs, cnt0_ref, cnt1_ref, w2_ref, b2_ref):
    inv = 1.0 / jnp.maximum(cnt0_ref[:, 0:1] + cnt1_ref[:, 0:1], 1.0)
    h = b2_ref[...]
    for g in range(2):
        h = h + jnp.dot(s_refs[g][...] * inv, w2_ref[g * _H:(g + 1) * _H, :],
                        precision=_PREC, preferred_element_type=jnp.float32)
    return h


def _tc_mid_body(s0, s1, cnt0_ref, cnt1_ref, w2_ref, b2_ref,
                 w1_ref, b1_ref, *t_refs):
    h = _agg_matmul((s0, s1), cnt0_ref, cnt1_ref, w2_ref, b2_ref)
    t = jnp.maximum(jnp.dot(h, w1_ref[...], precision=_PREC,
                            preferred_element_type=jnp.float32) + b1_ref[...],
                    0.0)
    _split_store(t, t_refs)


def _tc_dec_body(s0, s1, cnt0_ref, cnt1_ref, w2_ref, b2_ref,
                 wd_ref, bd_ref, out_ref):
    h = _agg_matmul((s0, s1), cnt0_ref, cnt1_ref, w2_ref, b2_ref)
    out_ref[...] = (jnp.dot(h, wd_ref[...], precision=_PREC,
                            preferred_element_type=jnp.float32) + bd_ref[...])


def _row_spec(w):
    return pl.BlockSpec((_BN, w), lambda i: (i, 0))


def _full_spec(shape):
    return pl.BlockSpec(shape, lambda i: tuple(0 for _ in shape))


_half_out = [jax.ShapeDtypeStruct((N, _H), jnp.float32)] * 2
_half_specs = [_row_spec(_H)] * 2

_tc_enc = pl.pallas_call(
    _tc_enc_body,
    grid=(N // _BN,),
    in_specs=[_row_spec(D_IN), _full_spec((D_IN, D_H)), _full_spec((1, D_H)),
              _full_spec((D_H, D_H)), _full_spec((1, D_H))],
    out_specs=_half_specs,
    out_shape=_half_out,
)

_tc_mid = pl.pallas_call(
    _tc_mid_body,
    grid=(N // _BN,),
    in_specs=_half_specs + [_row_spec(16), _row_spec(16),
              _full_spec((D_H, D_H)), _full_spec((1, D_H)),
              _full_spec((D_H, D_H)), _full_spec((1, D_H))],
    out_specs=_half_specs,
    out_shape=_half_out,
)

_tc_dec = pl.pallas_call(
    _tc_dec_body,
    grid=(N // _BN,),
    in_specs=_half_specs + [_row_spec(16), _row_spec(16),
              _full_spec((D_H, D_H)), _full_spec((1, D_H)),
              _full_spec((D_H, D_OUT)), _full_spec((1, D_OUT))],
    out_specs=_row_spec(D_OUT),
    out_shape=jax.ShapeDtypeStruct((N, D_OUT), jnp.float32),
)


def kernel(x, edge_index, W_enc, b_enc, W1, b1, W2, b2, W_dec, b_dec):
    pad = _EPAD - E
    src_p = jnp.concatenate([edge_index[0], jnp.zeros((pad,), jnp.int32)])
    dst_p = jnp.concatenate([edge_index[1], jnp.full((pad,), N, jnp.int32)])
    src2e = src_p.reshape(_EROWS2, _C2)
    dst2e = dst_p.reshape(_EROWS2, _C2)
    dst2c = dst_p.reshape(_CEROWS, _CCHUNK)
    be = b_enc.reshape(1, D_H)
    b1r = b1.reshape(1, D_H)
    b2r = b2.reshape(1, D_H)
    bdr = b_dec.reshape(1, D_OUT)

    cnt0, cnt1 = _sc_count(dst2c)
    t = _tc_enc(x, W_enc, be, W1, b1r)
    s0, s1 = _sc_scatter(t[0], t[1], src2e, dst2e)
    u = _tc_mid(s0, s1, cnt0, cnt1, W2, b2r, W1, b1r)
    v0, v1 = _sc_scatter(u[0], u[1], src2e, dst2e)
    return _tc_dec(v0, v1, cnt0, cnt1, W2, b2r, W_dec, bdr)
